# baseline (speedup 1.0000x reference)
import functools

import jax
import jax.numpy as jnp
from jax.experimental import pallas as pl
from jax.experimental.pallas import tpu as pltpu


def _round_up(n, m):
    return (n + m - 1) // m * m


def _mlp_kernel(x_ref, w1t_ref, b1_ref, w2_ref, b2_ref, w3t_ref, b3_ref,
                w4_ref, b4_ref, out_ref, acc_ref, *, nk):
    bf = jnp.bfloat16
    f32 = jnp.float32
    k = pl.program_id(1)
    part = jax.lax.dot_general(
        x_ref[...].astype(bf), w1t_ref[...].astype(bf),
        (((1,), (1,)), ((), ())), preferred_element_type=f32)

    @pl.when(k == 0)
    def _():
        acc_ref[...] = part

    @pl.when((k > 0) & (k < nk - 1))
    def _():
        acc_ref[...] = acc_ref[...] + part

    @pl.when(k == nk - 1)
    def _():
        h = acc_ref[...] + part + b1_ref[...]
        h = jnp.maximum(h, 0.0)
        h = jnp.dot(h.astype(bf), w2_ref[...].astype(bf),
                    preferred_element_type=f32) + b2_ref[...]
        h = jnp.maximum(h, 0.0)
        h = jax.lax.dot_general(
            h.astype(bf), w3t_ref[...].astype(bf),
            (((1,), (1,)), ((), ())), preferred_element_type=f32) + b3_ref[...]
        h = jnp.maximum(h, 0.0)
        out = jax.lax.dot_general(
            w4_ref[...].astype(bf), h.astype(bf),
            (((0,), (1,)), ((), ())), preferred_element_type=f32)
        out_ref[...] = out + jnp.transpose(b4_ref[...])


def kernel(x, w1, b1, w2, b2, w3, b3, w4, b4):
    B, F = x.shape
    out_features = w4.shape[1]
    H1 = w1.shape[1]

    TB = min(4096, _round_up(B, 8))
    B_pad = _round_up(B, TB)
    if B_pad != B:
        x = jnp.zeros((B_pad, F), x.dtype).at[:B].set(x)

    NK = 2 if F % 256 == 0 else 1
    FK = F // NK

    w1t = jnp.transpose(w1)
    w3t = jnp.transpose(w3)

    grid = (B_pad // TB, NK)
    flops = 2 * B_pad * (F * H1 + w2.size + w3.size + w4.size)
    bytes_accessed = 4 * (B_pad * F + w1.size + w2.size + w3.size + w4.size
                          + B_pad * out_features)

    whole = lambda shape: pl.BlockSpec(
        shape, lambda i, k: tuple(0 for _ in shape))

    out_t = pl.pallas_call(
        functools.partial(_mlp_kernel, nk=NK),
        out_shape=jax.ShapeDtypeStruct((out_features, B_pad), jnp.float32),
        grid=grid,
        in_specs=[
            pl.BlockSpec((TB, FK), lambda i, k: (i, k)),
            pl.BlockSpec((H1, FK), lambda i, k: (0, k)),
            whole(b1.shape),
            whole(w2.shape), whole(b2.shape),
            whole(w3t.shape), whole(b3.shape),
            whole(w4.shape), whole(b4.shape),
        ],
        out_specs=pl.BlockSpec((out_features, TB), lambda i, k: (0, i)),
        scratch_shapes=[pltpu.VMEM((TB, H1), jnp.float32)],
        compiler_params=pltpu.CompilerParams(
            dimension_semantics=("parallel", "arbitrary"),
            vmem_limit_bytes=64 * 1024 * 1024,
        ),
        cost_estimate=pl.CostEstimate(
            flops=flops, transcendentals=0, bytes_accessed=bytes_accessed),
    )(x, w1t, b1, w2, b2, w3t, b3, w4, b4)

    return out_t[:, :B].T


# speedup vs baseline: 1.0461x; 1.0461x over previous
import jax
import jax.numpy as jnp
from jax.experimental import pallas as pl
from jax.experimental.pallas import tpu as pltpu


def _round_up(n, m):
    return (n + m - 1) // m * m


def _mlp_kernel(x_ref, w1t_ref, b1_ref, w2_ref, b2_ref, w3t_ref, b3_ref,
                w4_ref, b4_ref, out_ref):
    bf = jnp.bfloat16
    f32 = jnp.float32
    h = jax.lax.dot_general(
        x_ref[...].astype(bf), w1t_ref[...].astype(bf),
        (((1,), (1,)), ((), ())), preferred_element_type=f32) + b1_ref[...]
    h = jnp.maximum(h, 0.0)
    h = jnp.dot(h.astype(bf), w2_ref[...].astype(bf),
                preferred_element_type=f32) + b2_ref[...]
    h = jnp.maximum(h, 0.0)
    h = jax.lax.dot_general(
        h.astype(bf), w3t_ref[...].astype(bf),
        (((1,), (1,)), ((), ())), preferred_element_type=f32) + b3_ref[...]
    h = jnp.maximum(h, 0.0)
    out = jax.lax.dot_general(
        w4_ref[...].astype(bf), h.astype(bf),
        (((0,), (1,)), ((), ())), preferred_element_type=f32)
    out_ref[...] = out + jnp.transpose(b4_ref[...])


def kernel(x, w1, b1, w2, b2, w3, b3, w4, b4):
    B, F = x.shape
    out_features = w4.shape[1]

    TB = min(4096, _round_up(B, 8))
    B_pad = _round_up(B, TB)
    if B_pad != B:
        x = jnp.zeros((B_pad, F), x.dtype).at[:B].set(x)

    w1t = jnp.transpose(w1)
    w3t = jnp.transpose(w3)

    grid = (B_pad // TB,)
    flops = 2 * B_pad * (F * w1.shape[1] + w2.size + w3.size + w4.size)
    bytes_accessed = 4 * (B_pad * F + w1.size + w2.size + w3.size + w4.size
                          + B_pad * out_features)

    whole = lambda shape: pl.BlockSpec(shape, lambda i: tuple(0 for _ in shape))

    out_t = pl.pallas_call(
        _mlp_kernel,
        out_shape=jax.ShapeDtypeStruct((out_features, B_pad), jnp.float32),
        grid=grid,
        in_specs=[
            pl.BlockSpec((TB, F), lambda i: (i, 0)),
            whole(w1t.shape), whole(b1.shape),
            whole(w2.shape), whole(b2.shape),
            whole(w3t.shape), whole(b3.shape),
            whole(w4.shape), whole(b4.shape),
        ],
        out_specs=pl.BlockSpec((out_features, TB), lambda i: (0, i)),
        compiler_params=pltpu.CompilerParams(
            dimension_semantics=("parallel",),
            vmem_limit_bytes=64 * 1024 * 1024,
        ),
        cost_estimate=pl.CostEstimate(
            flops=flops, transcendentals=0, bytes_accessed=bytes_accessed),
    )(x, w1t, b1, w2, b2, w3t, b3, w4, b4)

    return out_t[:, :B].T
